# baseline (speedup 1.0000x reference)
import functools
import math

import jax
import jax.numpy as jnp
from jax import lax
from jax.experimental import pallas as pl
from jax.experimental.pallas import tpu as pltpu
from jax.experimental.pallas import tpu_sc as plsc

S = 64.0
MARGIN = 0.5
_COS_M = math.cos(MARGIN)
_SIN_M = math.sin(MARGIN)

_NC = 2
_NS = 16
_NW = _NC * _NS
_L = 16
_CHUNK = 50000
_NBUF = 2


def _sqrt1mt2(t):
    z = jnp.maximum(1.0 - t * t, 0.0)
    s = jnp.full_like(z, 0.5)
    for _ in range(22):
        s = 0.5 * (s + z / s)
    return s


def _sc_body(n_rows, n_cols, logits_hbm, labels_hbm, out_hbm,
             labv, idxv, tv, valv, buf0, buf1,
             sin0, sin1, sout0, sout1, sfix):
    rows_per_w = n_rows // _NW
    span = rows_per_w * n_cols
    nchunks = span // _CHUNK
    wid = lax.axis_index("s") * _NC + lax.axis_index("c")
    base = wid * span
    bufs = (buf0, buf1)
    sins = (sin0, sin1)
    souts = (sout0, sout1)

    def copy_in(c, b):
        return pltpu.async_copy(
            logits_hbm.at[pl.ds(base + c * _CHUNK, _CHUNK)], bufs[b], sins[b])

    def copy_out(c, b):
        return pltpu.async_copy(
            bufs[b], out_hbm.at[pl.ds(base + c * _CHUNK, _CHUNK)], souts[b])

    outs = [None] * nchunks
    ins = [None] * nchunks
    for c in range(min(_NBUF, nchunks)):
        ins[c] = copy_in(c, c % _NBUF)
    for c in range(nchunks):
        b = c % _NBUF
        ins[c].wait()
        buf = bufs[b]

        @plsc.parallel_loop(0, _CHUNK // _L, unroll=8)
        def _scale(i, buf=buf):
            sl = pl.ds(i * _L, _L)
            buf[sl] = buf[sl] * S

        outs[c] = copy_out(c, b)
        nxt = c - 1 + _NBUF
        if c >= 1 and nxt < nchunks:
            outs[c - 1].wait()
            ins[nxt] = copy_in(nxt, (c - 1) % _NBUF)
    for c in range(max(0, nchunks - _NBUF), nchunks):
        outs[c].wait()

    row0 = wid * rows_per_w
    pltpu.sync_copy(labels_hbm.at[pl.ds(row0, rows_per_w)], labv)
    for h in range(rows_per_w // _L):
        lab = labv[pl.ds(h * _L, _L)]
        safe = jnp.where(lab >= 0, lab, 0)
        rows = row0 + h * _L + lax.broadcasted_iota(jnp.int32, (_L,), 0)
        idxv[pl.ds(h * _L, _L)] = rows * n_cols + safe
    pltpu.async_copy(logits_hbm.at[idxv], tv, sfix).wait()
    for h in range(rows_per_w // _L):
        t = tv[pl.ds(h * _L, _L)]
        lab = labv[pl.ds(h * _L, _L)]
        fixed = (t * _COS_M - _sqrt1mt2(t) * _SIN_M) * S
        valv[pl.ds(h * _L, _L)] = jnp.where(lab >= 0, fixed, t * S)
    pltpu.async_copy(valv, out_hbm.at[idxv], sfix).wait()


def kernel(logits, labels):
    n_rows, n_cols = logits.shape
    rows_per_w = n_rows // _NW
    logits_flat = logits.reshape(-1)
    mesh = plsc.VectorSubcoreMesh(
        core_axis_name="c", subcore_axis_name="s",
        num_cores=_NC, num_subcores=_NS)
    out_flat = pl.kernel(
        functools.partial(_sc_body, n_rows, n_cols),
        out_type=jax.ShapeDtypeStruct((n_rows * n_cols,), jnp.float32),
        mesh=mesh,
        scratch_types=[
            pltpu.VMEM((rows_per_w,), jnp.int32),
            pltpu.VMEM((rows_per_w,), jnp.int32),
            pltpu.VMEM((rows_per_w,), jnp.float32),
            pltpu.VMEM((rows_per_w,), jnp.float32),
            pltpu.VMEM((_CHUNK,), jnp.float32),
            pltpu.VMEM((_CHUNK,), jnp.float32),
            pltpu.SemaphoreType.DMA,
            pltpu.SemaphoreType.DMA,
            pltpu.SemaphoreType.DMA,
            pltpu.SemaphoreType.DMA,
            pltpu.SemaphoreType.DMA,
        ],
    )(logits_flat, labels)
    return out_flat.reshape(n_rows, n_cols)


# speedup vs baseline: 1.3341x; 1.3341x over previous
import functools
import math

import jax
import jax.numpy as jnp
from jax import lax
from jax.experimental import pallas as pl
from jax.experimental.pallas import tpu as pltpu
from jax.experimental.pallas import tpu_sc as plsc

S = 64.0
MARGIN = 0.5
_COS_M = math.cos(MARGIN)
_SIN_M = math.sin(MARGIN)

_NC = 2
_NS = 16
_NW = _NC * _NS
_L = 16

_BLOCK_COLS = 2048


def _sqrt1mt2(t):
    z = jnp.maximum(1.0 - t * t, 0.0)
    s = jnp.full_like(z, 0.5)
    for _ in range(22):
        s = 0.5 * (s + z / s)
    return s


def _sc_fix_body(n_rows, n_cols, logits_hbm, labels_hbm, fixed_hbm,
                 labv, idxv, tv, valv, sem):
    rows_per_w = n_rows // _NW
    wid = lax.axis_index("s") * _NC + lax.axis_index("c")
    row0 = wid * rows_per_w
    pltpu.sync_copy(labels_hbm.at[pl.ds(row0, rows_per_w)], labv)
    for h in range(rows_per_w // _L):
        lab = labv[pl.ds(h * _L, _L)]
        safe = jnp.where(lab >= 0, lab, 0)
        rows = row0 + h * _L + lax.broadcasted_iota(jnp.int32, (_L,), 0)
        idxv[pl.ds(h * _L, _L)] = rows * n_cols + safe
    pltpu.async_copy(logits_hbm.at[idxv], tv, sem).wait()
    for h in range(rows_per_w // _L):
        t = tv[pl.ds(h * _L, _L)]
        lab = labv[pl.ds(h * _L, _L)]
        fixed = (t * _COS_M - _sqrt1mt2(t) * _SIN_M) * S
        valv[pl.ds(h * _L, _L)] = jnp.where(lab >= 0, fixed, t * S)
    pltpu.sync_copy(valv, fixed_hbm.at[pl.ds(row0, rows_per_w)])


def _sc_fixed_values(logits_flat, labels, n_rows, n_cols):
    rows_per_w = n_rows // _NW
    mesh = plsc.VectorSubcoreMesh(
        core_axis_name="c", subcore_axis_name="s",
        num_cores=_NC, num_subcores=_NS)
    return pl.kernel(
        functools.partial(_sc_fix_body, n_rows, n_cols),
        out_type=jax.ShapeDtypeStruct((n_rows,), jnp.float32),
        mesh=mesh,
        scratch_types=[
            pltpu.VMEM((rows_per_w,), jnp.int32),
            pltpu.VMEM((rows_per_w,), jnp.int32),
            pltpu.VMEM((rows_per_w,), jnp.float32),
            pltpu.VMEM((rows_per_w,), jnp.float32),
            pltpu.SemaphoreType.DMA,
        ],
    )(logits_flat, labels)


def _tc_stream_body(logits_ref, labels_ref, fixed_ref, out_ref, *, block_cols):
    j = pl.program_id(0)
    x = logits_ref[...]
    rows, cols = x.shape
    col_ids = jax.lax.broadcasted_iota(jnp.int32, (rows, cols), 1) + j * block_cols
    mask = col_ids == labels_ref[...]
    out_ref[...] = jnp.where(mask, fixed_ref[...], x * S)


def kernel(logits, labels):
    n_rows, n_cols = logits.shape
    fixed = _sc_fixed_values(logits.reshape(-1), labels, n_rows, n_cols)
    grid = (pl.cdiv(n_cols, _BLOCK_COLS),)
    return pl.pallas_call(
        functools.partial(_tc_stream_body, block_cols=_BLOCK_COLS),
        grid=grid,
        in_specs=[
            pl.BlockSpec((n_rows, _BLOCK_COLS), lambda j: (0, j)),
            pl.BlockSpec((n_rows, 1), lambda j: (0, 0)),
            pl.BlockSpec((n_rows, 1), lambda j: (0, 0)),
        ],
        out_specs=pl.BlockSpec((n_rows, _BLOCK_COLS), lambda j: (0, j)),
        out_shape=jax.ShapeDtypeStruct((n_rows, n_cols), jnp.float32),
    )(logits, labels.reshape(n_rows, 1), fixed.reshape(n_rows, 1))


# speedup vs baseline: 8.1341x; 6.0971x over previous
import functools
import math

import jax
import jax.numpy as jnp
from jax import lax
from jax.experimental import pallas as pl

S = 64.0
MARGIN = 0.5
_COS_M = math.cos(MARGIN)
_SIN_M = math.sin(MARGIN)

_BLOCK_R = 2048


def _arcface_block_t(lt_ref, lab_ref, out_ref, *, block_r):
    j = pl.program_id(0)
    x = lt_ref[...]
    r, c = x.shape
    row_ids = lax.broadcasted_iota(jnp.int32, (r, c), 0) + j * block_r
    mask = row_ids == lab_ref[...]
    t = jnp.sum(jnp.where(mask, x, 0.0), axis=0, keepdims=True)
    sin_t = jnp.sqrt(jnp.maximum(1.0 - t * t, 0.0))
    fixed = (t * _COS_M - sin_t * _SIN_M) * S
    out_ref[...] = jnp.where(mask, fixed, x * S)


def kernel(logits, labels):
    n_rows, n_cols = logits.shape
    lt = logits.T
    labels_row = labels.reshape(1, n_rows)
    grid = (pl.cdiv(n_cols, _BLOCK_R),)
    out_t = pl.pallas_call(
        functools.partial(_arcface_block_t, block_r=_BLOCK_R),
        grid=grid,
        in_specs=[
            pl.BlockSpec((_BLOCK_R, n_rows), lambda j: (j, 0)),
            pl.BlockSpec((1, n_rows), lambda j: (0, 0)),
        ],
        out_specs=pl.BlockSpec((_BLOCK_R, n_rows), lambda j: (j, 0)),
        out_shape=jax.ShapeDtypeStruct((n_cols, n_rows), jnp.float32),
    )(lt, labels_row)
    return out_t.T
